# baseline (speedup 1.0000x reference)
import functools

import jax
import jax.numpy as jnp
from jax import lax
from jax.experimental import pallas as pl
from jax.experimental.pallas import tpu as pltpu
from jax.experimental.pallas import tpu_sc as plsc

_NLY = 2
_LANES = 16


def _adj_body(n, n_edges, rows_per_w, eye_hbm, src_hbm, dst_hbm, c_hbm,
              src_v, dst_v, cloc):
    wid = lax.axis_index("s") * 2 + lax.axis_index("c")
    base = wid * rows_per_w
    fbase = base * n
    nloc = rows_per_w * n
    pltpu.sync_copy(src_hbm, src_v)
    pltpu.sync_copy(dst_hbm, dst_v)
    pltpu.sync_copy(eye_hbm.at[pl.ds(fbase, nloc)], cloc)

    ones = jnp.ones((_LANES,), jnp.float32)

    def body(i, carry):
        s = src_v[pl.ds(i * _LANES, _LANES)]
        d = dst_v[pl.ds(i * _LANES, _LANES)]
        m = (d >= base) & (d < base + rows_per_w)
        idx = jnp.where(m, (d - base) * n + s, 0)
        plsc.addupdate_scatter(cloc, [idx], ones, mask=m)
        return carry

    lax.fori_loop(0, n_edges // _LANES, body, 0)
    pltpu.sync_copy(cloc, c_hbm.at[pl.ds(fbase, nloc)])


def _build_adj(n, src, dst):
    n_edges = src.shape[0]
    rows_per_w = n // 32
    mesh = plsc.VectorSubcoreMesh(core_axis_name="c", subcore_axis_name="s")
    eye = jnp.eye(n, dtype=jnp.float32).reshape(n * n)
    fn = pl.kernel(
        functools.partial(_adj_body, n, n_edges, rows_per_w),
        out_type=jax.ShapeDtypeStruct((n * n,), jnp.float32),
        mesh=mesh,
        scratch_types=[
            pltpu.VMEM((n_edges,), jnp.int32),
            pltpu.VMEM((n_edges,), jnp.int32),
            pltpu.VMEM((rows_per_w * n,), jnp.float32),
        ],
        compiler_params=pltpu.CompilerParams(needs_layout_passes=False),
    )
    return fn(eye, src, dst).reshape(n, n)


def _gat_one(h, c, w_ref, asrc_ref, adst_ref, b_ref):
    for i in range(_NLY):
        w = w_ref[i]
        xl = lax.dot_general(h.astype(jnp.bfloat16), w.astype(jnp.bfloat16),
                             (((1,), (1,)), ((), ())),
                             preferred_element_type=jnp.float32)
        att2 = jnp.concatenate([asrc_ref[i:i + 1], adst_ref[i:i + 1]],
                               axis=0)
        watt = lax.dot_general(att2, w, (((1,), (0,)), ((), ())),
                               preferred_element_type=jnp.float32)
        a_row = lax.dot_general(watt, h, (((1,), (1,)), ((), ())),
                                preferred_element_type=jnp.float32)
        a_col = lax.dot_general(h, watt, (((1,), (1,)), ((), ())),
                                preferred_element_type=jnp.float32)
        arow0 = a_row[0:1, :]
        acol1 = a_col[:, 1:2]
        zm = acol1 + jnp.max(arow0)
        am = jnp.maximum(zm, 0.2 * zm)
        z = acol1 + arow0
        al = jnp.maximum(z, 0.2 * z)
        ex = c * jnp.exp(al - am)
        den = jnp.sum(ex, axis=1, keepdims=True)
        out = jnp.dot(ex.astype(jnp.bfloat16), xl.astype(jnp.bfloat16),
                      preferred_element_type=jnp.float32)
        out = out / (den + 1e-16)
        h = jnp.maximum(out + b_ref[i][None, :], 0.0)
    return h


def _gat_body(rep, x_ref, e_ref, c_ref, w_ref, asrc_ref, adst_ref, b_ref,
              o_ref):
    c = c_ref[...]
    for r in range(rep):
        h = jnp.concatenate([x_ref[r], e_ref[r]], axis=0)
        o_ref[r] = _gat_one(h, c, w_ref, asrc_ref, adst_ref,
                            b_ref).astype(jnp.bfloat16)


def _run_gat(x3, e3, c, gat_W, gat_att_src, gat_att_dst, gat_bias, rep=16):
    bt, nx, d = x3.shape
    ne = e3.shape[1]
    n = nx + ne
    return pl.pallas_call(
        functools.partial(_gat_body, rep),
        grid=(bt // rep,),
        in_specs=[
            pl.BlockSpec((rep, nx, d), lambda i: (i, 0, 0)),
            pl.BlockSpec((rep, ne, d), lambda i: (i, 0, 0)),
            pl.BlockSpec((n, n), lambda i: (0, 0)),
            pl.BlockSpec((_NLY, d, d), lambda i: (0, 0, 0)),
            pl.BlockSpec((_NLY, d), lambda i: (0, 0)),
            pl.BlockSpec((_NLY, d), lambda i: (0, 0)),
            pl.BlockSpec((_NLY, d), lambda i: (0, 0)),
        ],
        out_specs=pl.BlockSpec((rep, n, d), lambda i: (i, 0, 0)),
        out_shape=jax.ShapeDtypeStruct((bt, n, d), jnp.bfloat16),
        compiler_params=pltpu.CompilerParams(
            dimension_semantics=("parallel",)),
    )(x3, e3, c, gat_W, gat_att_src, gat_att_dst, gat_bias)


def _conv_body(kern, t, nb, nxb, h_ref, w_ref, b_ref, ox_ref, oe_ref):
    h = h_ref[0]
    d = h.shape[-1]
    for i in range(_NLY):
        dil = 2 ** i
        hb = h.astype(jnp.bfloat16)
        acc = None
        for k in range(kern):
            s = (kern - 1 - k) * dil
            if s == 0:
                src = hb
            else:
                src = jnp.concatenate(
                    [jnp.zeros((s, nb, d), jnp.bfloat16), hb[: t - s]],
                    axis=0)
            y = jnp.dot(src.reshape(t * nb, d), w_ref[i, k],
                        preferred_element_type=jnp.float32)
            acc = y if acc is None else acc + y
        h = jnp.maximum(acc + b_ref[i][None, :], 0.0).reshape(t, nb, d)
    j = pl.program_id(1)

    @pl.when(j < nxb)
    def _():
        ox_ref[0] = h

    @pl.when(j == nxb)
    def _():
        oe_ref[0] = h


def _run_conv(h4, conv_W, conv_b, nx):
    b, t, n, d = h4.shape
    nb = 64
    kern = conv_W.shape[-1]
    nxb = nx // nb
    w_r = jnp.transpose(conv_W, (0, 3, 2, 1)).astype(jnp.bfloat16)
    return pl.pallas_call(
        functools.partial(_conv_body, kern, t, nb, nxb),
        grid=(b, n // nb),
        in_specs=[
            pl.BlockSpec((1, t, nb, d), lambda i, j: (i, 0, j, 0)),
            pl.BlockSpec((_NLY, kern, d, d), lambda i, j: (0, 0, 0, 0)),
            pl.BlockSpec((_NLY, d), lambda i, j: (0, 0)),
        ],
        out_specs=[
            pl.BlockSpec((1, t, nb, d),
                         lambda i, j: (i, 0, jnp.minimum(j, nxb - 1), 0)),
            pl.BlockSpec((1, t, nb, d), lambda i, j: (i, 0, 0, 0)),
        ],
        out_shape=[
            jax.ShapeDtypeStruct((b, t, nx, d), jnp.float32),
            jax.ShapeDtypeStruct((b, t, n - nx, d), jnp.float32),
        ],
        compiler_params=pltpu.CompilerParams(
            dimension_semantics=("parallel", "arbitrary")),
    )(h4, w_r, conv_b)


def kernel(x, e, edge_ind, gat_W, gat_att_src, gat_att_dst, gat_bias,
           conv_W, conv_b):
    b, t, nx, d = x.shape
    ne = e.shape[2]
    n = nx + ne
    ei = edge_ind.astype(jnp.int32)
    c = _build_adj(n, ei[0], ei[1])
    hf = _run_gat(x.reshape(b * t, nx, d), e.reshape(b * t, ne, d), c,
                  gat_W, gat_att_src, gat_att_dst, gat_bias)
    h4 = hf.reshape(b, t, n, d)
    x_out, e_out = _run_conv(h4, conv_W, conv_b, nx)
    return x_out, e_out


# speedup vs baseline: 1.0223x; 1.0223x over previous
import functools

import jax
import jax.numpy as jnp
from jax import lax
from jax.experimental import pallas as pl
from jax.experimental.pallas import tpu as pltpu
from jax.experimental.pallas import tpu_sc as plsc

_NLY = 2
_LANES = 16


def _adj_body(n, n_edges, rows_per_w, eye_hbm, src_hbm, dst_hbm, c_hbm,
              src_v, dst_v, cloc):
    wid = lax.axis_index("s") * 2 + lax.axis_index("c")
    base = wid * rows_per_w
    fbase = base * n
    nloc = rows_per_w * n
    pltpu.sync_copy(src_hbm, src_v)
    pltpu.sync_copy(dst_hbm, dst_v)
    pltpu.sync_copy(eye_hbm.at[pl.ds(fbase, nloc)], cloc)

    ones = jnp.ones((_LANES,), jnp.float32)

    def body(i, carry):
        s = src_v[pl.ds(i * _LANES, _LANES)]
        d = dst_v[pl.ds(i * _LANES, _LANES)]
        m = (d >= base) & (d < base + rows_per_w)
        idx = jnp.where(m, (d - base) * n + s, 0)
        plsc.addupdate_scatter(cloc, [idx], ones, mask=m)
        return carry

    lax.fori_loop(0, n_edges // _LANES, body, 0)
    pltpu.sync_copy(cloc, c_hbm.at[pl.ds(fbase, nloc)])


def _build_adj(n, src, dst):
    n_edges = src.shape[0]
    rows_per_w = n // 32
    mesh = plsc.VectorSubcoreMesh(core_axis_name="c", subcore_axis_name="s")
    eye = jnp.eye(n, dtype=jnp.float32).reshape(n * n)
    fn = pl.kernel(
        functools.partial(_adj_body, n, n_edges, rows_per_w),
        out_type=jax.ShapeDtypeStruct((n * n,), jnp.float32),
        mesh=mesh,
        scratch_types=[
            pltpu.VMEM((n_edges,), jnp.int32),
            pltpu.VMEM((n_edges,), jnp.int32),
            pltpu.VMEM((rows_per_w * n,), jnp.float32),
        ],
        compiler_params=pltpu.CompilerParams(needs_layout_passes=False),
    )
    return fn(eye, src, dst).reshape(n, n)


def _gat_one(h, c, w_ref, asrc_ref, adst_ref, b_ref):
    for i in range(_NLY):
        w = w_ref[i]
        xl = lax.dot_general(h.astype(jnp.bfloat16), w.astype(jnp.bfloat16),
                             (((1,), (1,)), ((), ())),
                             preferred_element_type=jnp.float32)
        att2 = jnp.concatenate([asrc_ref[i:i + 1], adst_ref[i:i + 1]],
                               axis=0)
        watt = lax.dot_general(att2, w, (((1,), (0,)), ((), ())),
                               preferred_element_type=jnp.float32)
        a_row = lax.dot_general(watt, h, (((1,), (1,)), ((), ())),
                                preferred_element_type=jnp.float32)
        a_col = lax.dot_general(h, watt, (((1,), (1,)), ((), ())),
                                preferred_element_type=jnp.float32)
        arow0 = a_row[0:1, :]
        acol1 = a_col[:, 1:2]
        zm = acol1 + jnp.max(arow0)
        am = jnp.maximum(zm, 0.2 * zm)
        z = acol1 + arow0
        al = jnp.maximum(z, 0.2 * z)
        ex = c * jnp.exp(al - am)
        den = jnp.sum(ex, axis=1, keepdims=True)
        out = jnp.dot(ex.astype(jnp.bfloat16), xl.astype(jnp.bfloat16),
                      preferred_element_type=jnp.float32)
        out = out / (den + 1e-16)
        h = jnp.maximum(out + b_ref[i][None, :], 0.0)
    return h


def _gat_body(rep, x_ref, e_ref, c_ref, w_ref, asrc_ref, adst_ref, b_ref,
              o_ref):
    c = c_ref[...]
    for r in range(rep):
        h = jnp.concatenate([x_ref[r], e_ref[r]], axis=0)
        o_ref[r] = _gat_one(h, c, w_ref, asrc_ref, adst_ref,
                            b_ref).astype(jnp.bfloat16)


def _run_gat(x3, e3, c, gat_W, gat_att_src, gat_att_dst, gat_bias, rep=16):
    bt, nx, d = x3.shape
    ne = e3.shape[1]
    n = nx + ne
    return pl.pallas_call(
        functools.partial(_gat_body, rep),
        grid=(bt // rep,),
        in_specs=[
            pl.BlockSpec((rep, nx, d), lambda i: (i, 0, 0)),
            pl.BlockSpec((rep, ne, d), lambda i: (i, 0, 0)),
            pl.BlockSpec((n, n), lambda i: (0, 0)),
            pl.BlockSpec((_NLY, d, d), lambda i: (0, 0, 0)),
            pl.BlockSpec((_NLY, d), lambda i: (0, 0)),
            pl.BlockSpec((_NLY, d), lambda i: (0, 0)),
            pl.BlockSpec((_NLY, d), lambda i: (0, 0)),
        ],
        out_specs=pl.BlockSpec((rep, n, d), lambda i: (i, 0, 0)),
        out_shape=jax.ShapeDtypeStruct((bt, n, d), jnp.bfloat16),
        compiler_params=pltpu.CompilerParams(
            dimension_semantics=("arbitrary",)),
    )(x3, e3, c, gat_W, gat_att_src, gat_att_dst, gat_bias)


def _conv_body(kern, t, nb, nxb, bb, h_ref, w_ref, b_ref, ox_ref, oe_ref):
    d = h_ref.shape[-1]
    j = pl.program_id(1)
    for r in range(bb):
        h = h_ref[r]
        for i in range(_NLY):
            dil = 2 ** i
            hb = h.astype(jnp.bfloat16)
            acc = None
            for k in range(kern):
                s = (kern - 1 - k) * dil
                if s == 0:
                    src = hb
                else:
                    src = jnp.concatenate(
                        [jnp.zeros((s, nb, d), jnp.bfloat16), hb[: t - s]],
                        axis=0)
                y = jnp.dot(src.reshape(t * nb, d), w_ref[i, k],
                            preferred_element_type=jnp.float32)
                acc = y if acc is None else acc + y
            h = jnp.maximum(acc + b_ref[i][None, :], 0.0).reshape(t, nb, d)

        @pl.when(j < nxb)
        def _():
            ox_ref[r] = h

        @pl.when(j == nxb)
        def _():
            oe_ref[r] = h


def _run_conv(h4, conv_W, conv_b, nx, bb=2):
    b, t, n, d = h4.shape
    nb = 64
    kern = conv_W.shape[-1]
    nxb = nx // nb
    w_r = jnp.transpose(conv_W, (0, 3, 2, 1)).astype(jnp.bfloat16)
    return pl.pallas_call(
        functools.partial(_conv_body, kern, t, nb, nxb, bb),
        grid=(b // bb, n // nb),
        in_specs=[
            pl.BlockSpec((bb, t, nb, d), lambda i, j: (i, 0, j, 0)),
            pl.BlockSpec((_NLY, kern, d, d), lambda i, j: (0, 0, 0, 0)),
            pl.BlockSpec((_NLY, d), lambda i, j: (0, 0)),
        ],
        out_specs=[
            pl.BlockSpec((bb, t, nb, d),
                         lambda i, j: (i, 0, jnp.minimum(j, nxb - 1), 0)),
            pl.BlockSpec((bb, t, nb, d), lambda i, j: (i, 0, 0, 0)),
        ],
        out_shape=[
            jax.ShapeDtypeStruct((b, t, nx, d), jnp.float32),
            jax.ShapeDtypeStruct((b, t, n - nx, d), jnp.float32),
        ],
        compiler_params=pltpu.CompilerParams(
            dimension_semantics=("parallel", "arbitrary")),
    )(h4, w_r, conv_b)


def kernel(x, e, edge_ind, gat_W, gat_att_src, gat_att_dst, gat_bias,
           conv_W, conv_b):
    b, t, nx, d = x.shape
    ne = e.shape[2]
    n = nx + ne
    ei = edge_ind.astype(jnp.int32)
    c = _build_adj(n, ei[0], ei[1])
    hf = _run_gat(x.reshape(b * t, nx, d), e.reshape(b * t, ne, d), c,
                  gat_W, gat_att_src, gat_att_dst, gat_bias)
    h4 = hf.reshape(b, t, n, d)
    x_out, e_out = _run_conv(h4, conv_W, conv_b, nx)
    return x_out, e_out
